# Initial kernel scaffold; baseline (speedup 1.0000x reference)
import jax
import jax.numpy as jnp
from jax.experimental import pallas as pl


def kernel(u, i, user_table, item_feat_table, item_df_idx):
    raise NotImplementedError("write your pallas kernel here")


# speedup vs baseline: 2.1293x; 2.1293x over previous
import functools

import jax
import jax.numpy as jnp
from jax import lax
from jax.experimental import pallas as pl
from jax.experimental.pallas import tpu as pltpu
from jax.experimental.pallas import tpu_sc as plsc

NFEAT = 26
DIM = 32
EPS = 1e-7

NC = 2
NS = 16
NW = NC * NS
L = 16


def _norm_table_body(t_ref, o_ref):
    x = t_ref[...]
    n = jnp.sqrt(jnp.sum(x * x, axis=1, keepdims=True))
    s = jnp.minimum(1.0, 1.0 / (n + EPS))
    o_ref[...] = x * s


def _normalize_table(table):
    V, D = table.shape
    blk = 2000
    return pl.pallas_call(
        _norm_table_body,
        grid=(V // blk,),
        in_specs=[pl.BlockSpec((blk, D), lambda b: (b, 0))],
        out_specs=pl.BlockSpec((blk, D), lambda b: (b, 0)),
        out_shape=jax.ShapeDtypeStruct((V, D), jnp.float32),
    )(table)


_DIV26_M = 20165
_DIV26_S = 19


def _sc_body(i_hbm, u_hbm, dfflat_hbm, feat_hbm, user_hbm,
             sum_hbm, sq_hbm, urow_hbm,
             i_v, u_v, dfaddr_v, dfidx_v, rows_v, urows_v, sums_v, sq_v,
             sem_b, sem_c, *, b_per_w, chunk):
    wid = lax.axis_index("s") * NC + lax.axis_index("c")
    nchunks = b_per_w // chunk
    npos = chunk * NFEAT
    ngroups = npos // L
    npieces = npos // 128
    iota = lax.iota(jnp.int32, L)

    for k in range(nchunks):
        base = wid * b_per_w + k * chunk
        pltpu.sync_copy(i_hbm.at[pl.ds(base, chunk)], i_v)
        pltpu.sync_copy(u_hbm.at[pl.ds(base, chunk)], u_v)
        cp_usr = pltpu.async_copy(user_hbm.at[u_v], urows_v, sem_b)

        def g_body(g, carry):
            p = g * L + iota
            c = lax.shift_right_logical(p * _DIV26_M, _DIV26_S)
            f = p - c * NFEAT
            ivals = plsc.load_gather(i_v, [c])
            dfaddr_v[pl.ds(g * L, L)] = ivals * NFEAT + f
            return carry

        lax.fori_loop(0, ngroups, g_body, 0)

        for j in range(npieces):
            pltpu.make_async_copy(
                dfflat_hbm.at[dfaddr_v.at[pl.ds(j * 128, 128)]],
                dfidx_v.at[pl.ds(j * 128, 128)], sem_c).start()
        for j in range(npieces):
            pltpu.make_async_copy(
                dfflat_hbm.at[dfaddr_v.at[pl.ds(j * 128, 128)]],
                dfidx_v.at[pl.ds(j * 128, 128)], sem_c).wait()

        for j in range(npieces):
            pltpu.make_async_copy(
                feat_hbm.at[dfidx_v.at[pl.ds(j * 128, 128)]],
                rows_v.at[pl.ds(j * 128, 128)], sem_c).start()
        cp_usr.wait()
        pltpu.sync_copy(urows_v, urow_hbm.at[pl.ds(base, chunk)])
        for j in range(npieces):
            pltpu.make_async_copy(
                feat_hbm.at[dfidx_v.at[pl.ds(j * 128, 128)]],
                rows_v.at[pl.ds(j * 128, 128)], sem_c).wait()

        def c_body(c, carry):
            r = c * NFEAT
            acc0 = jnp.zeros((L,), jnp.float32)
            acc1 = jnp.zeros((L,), jnp.float32)
            sq = jnp.zeros((L,), jnp.float32)
            for f in range(NFEAT):
                x0 = rows_v[r + f, 0:L]
                x1 = rows_v[r + f, L:DIM]
                acc0 = acc0 + x0
                acc1 = acc1 + x1
                sq = sq + x0 * x0
                sq = sq + x1 * x1
            sums_v[c, 0:L] = acc0
            sums_v[c, L:DIM] = acc1
            sq_v[c, 0:L] = sq
            return carry

        lax.fori_loop(0, chunk, c_body, 0)
        pltpu.sync_copy(sums_v, sum_hbm.at[pl.ds(base, chunk)])
        pltpu.sync_copy(sq_v, sq_hbm.at[pl.ds(base, chunk)])


def _sc_gather(i, u, df, feat_n, user_table):
    B = i.shape[0]
    b_per_w = B // NW
    chunk = 64
    mesh = plsc.VectorSubcoreMesh(core_axis_name="c", subcore_axis_name="s")
    kern = functools.partial(
        pl.kernel,
        mesh=mesh,
        compiler_params=pltpu.CompilerParams(
            use_tc_tiling_on_sc=False, needs_layout_passes=False),
        out_type=[
            jax.ShapeDtypeStruct((B, DIM), jnp.float32),
            jax.ShapeDtypeStruct((B, L), jnp.float32),
            jax.ShapeDtypeStruct((B, DIM), jnp.float32),
        ],
        scratch_types=[
            pltpu.VMEM((chunk,), jnp.int32),
            pltpu.VMEM((chunk,), jnp.int32),
            pltpu.VMEM((chunk * NFEAT,), jnp.int32),
            pltpu.VMEM((chunk * NFEAT,), jnp.int32),
            pltpu.VMEM((chunk * NFEAT, DIM), jnp.float32),
            pltpu.VMEM((chunk, DIM), jnp.float32),
            pltpu.VMEM((chunk, DIM), jnp.float32),
            pltpu.VMEM((chunk, L), jnp.float32),
            pltpu.SemaphoreType.DMA,
            pltpu.SemaphoreType.DMA,
        ],
    )(functools.partial(_sc_body, b_per_w=b_per_w, chunk=chunk))
    return kern(i, u, df.reshape(-1), feat_n, user_table)


def _finish_body(sums_ref, sq_ref, urow_ref, o_ref):
    uraw = urow_ref[...]
    n2 = jnp.sum(uraw * uraw, axis=1, keepdims=True)
    s = jnp.minimum(1.0, 1.0 / (jnp.sqrt(n2) + EPS))
    uh = uraw * s
    v = sums_ref[...] + uh
    t = (jnp.sum(v * v, axis=1)
         - jnp.sum(sq_ref[...], axis=1)
         - jnp.sum(uh * uh, axis=1))
    o_ref[...] = jax.nn.sigmoid(t)


def _finish(sums, sq, urows):
    B = sums.shape[0]
    blk = 2048
    return pl.pallas_call(
        _finish_body,
        grid=(B // blk,),
        in_specs=[
            pl.BlockSpec((blk, DIM), lambda b: (b, 0)),
            pl.BlockSpec((blk, L), lambda b: (b, 0)),
            pl.BlockSpec((blk, DIM), lambda b: (b, 0)),
        ],
        out_specs=pl.BlockSpec((blk,), lambda b: (b,)),
        out_shape=jax.ShapeDtypeStruct((B,), jnp.float32),
    )(sums, sq, urows)


def kernel(u, i, user_table, item_feat_table, item_df_idx):
    feat_n = _normalize_table(item_feat_table)
    sums, sq, urows = _sc_gather(i, u, item_df_idx, feat_n, user_table)
    return _finish(sums, sq, urows)


# speedup vs baseline: 2.2393x; 1.0516x over previous
import functools

import jax
import jax.numpy as jnp
from jax import lax
from jax.experimental import pallas as pl
from jax.experimental.pallas import tpu as pltpu
from jax.experimental.pallas import tpu_sc as plsc

NFEAT = 26
DIM = 32
EPS = 1e-7

NC = 2
NS = 16
NW = NC * NS
L = 16

FLBLK = 16384


def _cdiv(a, b):
    return -(-a // b)


def _flat_pack_body(x_ref, o_ref, *, din):
    x = x_ref[...]
    if din < DIM:
        x = jnp.concatenate(
            [x, jnp.zeros((DIM - din, FLBLK), x.dtype)], axis=0)
    x3 = x.T.reshape(FLBLK // 4, 4, DIM)
    o_ref[...] = jnp.concatenate([x3[:, m, :] for m in range(4)], axis=1)


def _flatten_rows(table_t):
    D, N = table_t.shape
    g = _cdiv(N, FLBLK)
    out = pl.pallas_call(
        functools.partial(_flat_pack_body, din=D),
        grid=(g,),
        in_specs=[pl.BlockSpec((D, FLBLK), lambda b: (0, b))],
        out_specs=pl.BlockSpec((FLBLK // 4, 128), lambda b: (b, 0)),
        out_shape=jax.ShapeDtypeStruct((g * FLBLK // 4, 128), table_t.dtype),
    )(table_t)
    return out.reshape(g * FLBLK, DIM)


def _norm_flat_body(x_ref, o_ref):
    x = x_ref[...]
    n2 = jnp.sum(x * x, axis=0, keepdims=True)
    s = jnp.minimum(1.0, 1.0 / (jnp.sqrt(n2) + EPS))
    x3 = (x * s).T.reshape(FLBLK // 4, 4, DIM)
    o_ref[...] = jnp.concatenate([x3[:, m, :] for m in range(4)], axis=1)


def _norm_flatten(table_t):
    D, N = table_t.shape
    g = _cdiv(N, FLBLK)
    out = pl.pallas_call(
        _norm_flat_body,
        grid=(g,),
        in_specs=[pl.BlockSpec((D, FLBLK), lambda b: (0, b))],
        out_specs=pl.BlockSpec((FLBLK // 4, 128), lambda b: (b, 0)),
        out_shape=jax.ShapeDtypeStruct((g * FLBLK // 4, 128), jnp.float32),
    )(table_t)
    return out.reshape(g * FLBLK, DIM)


_DIV26_M = 20165
_DIV26_S = 19


def _sc_body(i_hbm, u_hbm, df_hbm, feat_hbm, user_hbm,
             sum_hbm, sq_hbm, urow_hbm,
             i_v, u_v, ids_v, dfidx_v, rows_v, urows_v, sums_v, sq_v,
             sem_a, sem_b, sem_c, *, b_per_w, chunk):
    wid = lax.axis_index("s") * NC + lax.axis_index("c")
    nchunks = b_per_w // chunk
    npos = chunk * NFEAT
    ngroups = npos // L
    npieces = npos // 128
    iota = lax.iota(jnp.int32, L)

    for k in range(nchunks):
        base = wid * b_per_w + k * chunk
        pltpu.sync_copy(i_hbm.at[pl.ds(base, chunk)], i_v)
        pltpu.sync_copy(u_hbm.at[pl.ds(base, chunk)], u_v)
        cp_ids = pltpu.async_copy(df_hbm.at[i_v], ids_v, sem_a)
        cp_usr = pltpu.async_copy(user_hbm.at[u_v], urows_v, sem_b)
        cp_ids.wait()

        def g_body(g, carry):
            p = g * L + iota
            c = lax.shift_right_logical(p * _DIV26_M, _DIV26_S)
            f = p - c * NFEAT
            dfidx_v[pl.ds(g * L, L)] = plsc.load_gather(ids_v, [c, f])
            return carry

        lax.fori_loop(0, ngroups, g_body, 0)

        for j in range(npieces):
            pltpu.make_async_copy(
                feat_hbm.at[dfidx_v.at[pl.ds(j * 128, 128)]],
                rows_v.at[pl.ds(j * 128, 128)], sem_c).start()
        cp_usr.wait()
        pltpu.sync_copy(urows_v, urow_hbm.at[pl.ds(base, chunk)])
        for j in range(npieces):
            pltpu.make_async_copy(
                feat_hbm.at[dfidx_v.at[pl.ds(j * 128, 128)]],
                rows_v.at[pl.ds(j * 128, 128)], sem_c).wait()

        def c_body(c, carry):
            r = c * NFEAT
            acc0 = jnp.zeros((L,), jnp.float32)
            acc1 = jnp.zeros((L,), jnp.float32)
            sq = jnp.zeros((L,), jnp.float32)
            for f in range(NFEAT):
                x0 = rows_v[r + f, 0:L]
                x1 = rows_v[r + f, L:DIM]
                acc0 = acc0 + x0
                acc1 = acc1 + x1
                sq = sq + x0 * x0
                sq = sq + x1 * x1
            sums_v[c, 0:L] = acc0
            sums_v[c, L:DIM] = acc1
            sq_v[c, 0:L] = sq
            return carry

        lax.fori_loop(0, chunk, c_body, 0)
        pltpu.sync_copy(sums_v, sum_hbm.at[pl.ds(base, chunk)])
        pltpu.sync_copy(sq_v, sq_hbm.at[pl.ds(base, chunk)])


def _sc_gather(i, u, df2d, feat2d, user2d):
    B = i.shape[0]
    b_per_w = B // NW
    chunk = 64
    mesh = plsc.VectorSubcoreMesh(core_axis_name="c", subcore_axis_name="s")
    kern = functools.partial(
        pl.kernel,
        mesh=mesh,
        compiler_params=pltpu.CompilerParams(
            use_tc_tiling_on_sc=False, needs_layout_passes=False),
        out_type=[
            jax.ShapeDtypeStruct((B, DIM), jnp.float32),
            jax.ShapeDtypeStruct((B, L), jnp.float32),
            jax.ShapeDtypeStruct((B, DIM), jnp.float32),
        ],
        scratch_types=[
            pltpu.VMEM((chunk,), jnp.int32),
            pltpu.VMEM((chunk,), jnp.int32),
            pltpu.VMEM((chunk, DIM), jnp.int32),
            pltpu.VMEM((chunk * NFEAT,), jnp.int32),
            pltpu.VMEM((chunk * NFEAT, DIM), jnp.float32),
            pltpu.VMEM((chunk, DIM), jnp.float32),
            pltpu.VMEM((chunk, DIM), jnp.float32),
            pltpu.VMEM((chunk, L), jnp.float32),
            pltpu.SemaphoreType.DMA,
            pltpu.SemaphoreType.DMA,
            pltpu.SemaphoreType.DMA,
        ],
    )(functools.partial(_sc_body, b_per_w=b_per_w, chunk=chunk))
    return kern(i, u, df2d, feat2d, user2d)


def _finish_body(sums_ref, sq_ref, urow_ref, o_ref):
    uraw = urow_ref[...]
    n2 = jnp.sum(uraw * uraw, axis=1, keepdims=True)
    s = jnp.minimum(1.0, 1.0 / (jnp.sqrt(n2) + EPS))
    uh = uraw * s
    v = sums_ref[...] + uh
    t = (jnp.sum(v * v, axis=1)
         - jnp.sum(sq_ref[...], axis=1)
         - jnp.sum(uh * uh, axis=1))
    o_ref[...] = jax.nn.sigmoid(t)


def _finish(sums, sq, urows):
    B = sums.shape[0]
    blk = 2048
    return pl.pallas_call(
        _finish_body,
        grid=(B // blk,),
        in_specs=[
            pl.BlockSpec((blk, DIM), lambda b: (b, 0)),
            pl.BlockSpec((blk, L), lambda b: (b, 0)),
            pl.BlockSpec((blk, DIM), lambda b: (b, 0)),
        ],
        out_specs=pl.BlockSpec((blk,), lambda b: (b,)),
        out_shape=jax.ShapeDtypeStruct((B,), jnp.float32),
    )(sums, sq, urows)


def kernel(u, i, user_table, item_feat_table, item_df_idx):
    user2d = _flatten_rows(user_table.T)
    df_bits = lax.bitcast_convert_type(item_df_idx, jnp.float32).T
    df2d = lax.bitcast_convert_type(_flatten_rows(df_bits), jnp.int32)
    feat2d = _norm_flatten(item_feat_table.T)
    sums, sq, urows = _sc_gather(i, u, df2d, feat2d, user2d)
    return _finish(sums, sq, urows)


# speedup vs baseline: 6.4693x; 2.8890x over previous
import functools

import jax
import jax.numpy as jnp
from jax import lax
from jax.experimental import pallas as pl
from jax.experimental.pallas import tpu as pltpu
from jax.experimental.pallas import tpu_sc as plsc

NFEAT = 26
DIM = 32
EPS = 1e-7

NC = 2
NS = 16
NW = NC * NS
L = 16

FLBLK = 16384


def _cdiv(a, b):
    return -(-a // b)


def _detile_body(x_ref, o_ref, *, din):
    o_ref[...] = x_ref[...].reshape(din, FLBLK // 128, 128)


def _detile_flat(table_t):
    D, N = table_t.shape
    g = _cdiv(N, FLBLK)
    out = pl.pallas_call(
        functools.partial(_detile_body, din=D),
        grid=(g,),
        in_specs=[pl.BlockSpec((D, FLBLK), lambda b: (0, b))],
        out_specs=pl.BlockSpec((D, FLBLK // 128, 128), lambda b: (0, b, 0)),
        out_shape=jax.ShapeDtypeStruct((D, g * FLBLK // 128, 128),
                                       table_t.dtype),
    )(table_t)
    return out.reshape(D * g * FLBLK), g * FLBLK


def _norm_flat_body(x_ref, o_ref):
    x = x_ref[...]
    n2 = jnp.sum(x * x, axis=0, keepdims=True)
    s = jnp.minimum(1.0, 1.0 / (jnp.sqrt(n2) + EPS))
    x3 = (x * s).T.reshape(FLBLK // 4, 4, DIM)
    o_ref[...] = jnp.concatenate([x3[:, m, :] for m in range(4)], axis=1)


def _norm_flatten(table_t):
    D, N = table_t.shape
    g = _cdiv(N, FLBLK)
    out = pl.pallas_call(
        _norm_flat_body,
        grid=(g,),
        in_specs=[pl.BlockSpec((D, FLBLK), lambda b: (0, b))],
        out_specs=pl.BlockSpec((FLBLK // 4, 128), lambda b: (b, 0)),
        out_shape=jax.ShapeDtypeStruct((g * FLBLK // 4, 128), jnp.float32),
    )(table_t)
    return out.reshape(g * FLBLK, DIM)


_DIV26_M = 20165
_DIV26_S = 19


def _sc_body(i_hbm, u_hbm, dfflat_hbm, feat_hbm, userflat_hbm,
             sum_hbm, sq_hbm, urow_hbm,
             i_v, u_v, dfaddr_v, dfidx_v, rows_v, uaddr_v, urows_v,
             sums_v, sq_v,
             sem_b, sem_c, *, b_per_w, chunk, npad_u, npad_df):
    wid = lax.axis_index("s") * NC + lax.axis_index("c")
    nchunks = b_per_w // chunk
    npos = chunk * NFEAT
    ngroups = npos // L
    npieces = npos // 128
    nupieces = chunk * DIM // 128
    iota = lax.iota(jnp.int32, L)
    ulo = iota * npad_u
    uhi = (iota + L) * npad_u

    for k in range(nchunks):
        base = wid * b_per_w + k * chunk
        pltpu.sync_copy(i_hbm.at[pl.ds(base, chunk)], i_v)
        pltpu.sync_copy(u_hbm.at[pl.ds(base, chunk)], u_v)

        def g_body(g, carry):
            p = g * L + iota
            c = lax.shift_right_logical(p * _DIV26_M, _DIV26_S)
            f = p - c * NFEAT
            ivals = plsc.load_gather(i_v, [c])
            dfaddr_v[pl.ds(g * L, L)] = f * npad_df + ivals
            return carry

        lax.fori_loop(0, ngroups, g_body, 0)
        for j in range(npieces):
            pltpu.make_async_copy(
                dfflat_hbm.at[dfaddr_v.at[pl.ds(j * 128, 128)]],
                dfidx_v.at[pl.ds(j * 128, 128)], sem_c).start()

        def uaddr_body(c, carry):
            uvals = plsc.load_gather(u_v, [jnp.zeros((L,), jnp.int32) + c])
            uaddr_v[pl.ds(c * DIM, L)] = ulo + uvals
            uaddr_v[pl.ds(c * DIM + L, L)] = uhi + uvals
            return carry

        lax.fori_loop(0, chunk, uaddr_body, 0)
        for j in range(nupieces):
            pltpu.make_async_copy(
                userflat_hbm.at[uaddr_v.at[pl.ds(j * 128, 128)]],
                urows_v.at[pl.ds(j * 128, 128)], sem_b).start()

        for j in range(npieces):
            pltpu.make_async_copy(
                dfflat_hbm.at[dfaddr_v.at[pl.ds(j * 128, 128)]],
                dfidx_v.at[pl.ds(j * 128, 128)], sem_c).wait()

        for j in range(npieces):
            pltpu.make_async_copy(
                feat_hbm.at[dfidx_v.at[pl.ds(j * 128, 128)]],
                rows_v.at[pl.ds(j * 128, 128)], sem_c).start()
        for j in range(nupieces):
            pltpu.make_async_copy(
                userflat_hbm.at[uaddr_v.at[pl.ds(j * 128, 128)]],
                urows_v.at[pl.ds(j * 128, 128)], sem_b).wait()
        pltpu.sync_copy(urows_v, urow_hbm.at[pl.ds(base * DIM, chunk * DIM)])
        for j in range(npieces):
            pltpu.make_async_copy(
                feat_hbm.at[dfidx_v.at[pl.ds(j * 128, 128)]],
                rows_v.at[pl.ds(j * 128, 128)], sem_c).wait()

        def c_body(c, carry):
            r = c * NFEAT
            acc0 = jnp.zeros((L,), jnp.float32)
            acc1 = jnp.zeros((L,), jnp.float32)
            sq = jnp.zeros((L,), jnp.float32)
            for f in range(NFEAT):
                x0 = rows_v[r + f, 0:L]
                x1 = rows_v[r + f, L:DIM]
                acc0 = acc0 + x0
                acc1 = acc1 + x1
                sq = sq + x0 * x0
                sq = sq + x1 * x1
            sums_v[c, 0:L] = acc0
            sums_v[c, L:DIM] = acc1
            sq_v[c, 0:L] = sq
            return carry

        lax.fori_loop(0, chunk, c_body, 0)
        pltpu.sync_copy(sums_v, sum_hbm.at[pl.ds(base, chunk)])
        pltpu.sync_copy(sq_v, sq_hbm.at[pl.ds(base, chunk)])


def _sc_gather(i, u, df_flat, feat2d, user_flat, npad_u, npad_df):
    B = i.shape[0]
    b_per_w = B // NW
    chunk = 64
    mesh = plsc.VectorSubcoreMesh(core_axis_name="c", subcore_axis_name="s")
    kern = functools.partial(
        pl.kernel,
        mesh=mesh,
        compiler_params=pltpu.CompilerParams(
            use_tc_tiling_on_sc=False, needs_layout_passes=False),
        out_type=[
            jax.ShapeDtypeStruct((B, DIM), jnp.float32),
            jax.ShapeDtypeStruct((B, L), jnp.float32),
            jax.ShapeDtypeStruct((B * DIM,), jnp.float32),
        ],
        scratch_types=[
            pltpu.VMEM((chunk,), jnp.int32),
            pltpu.VMEM((chunk,), jnp.int32),
            pltpu.VMEM((chunk * NFEAT,), jnp.int32),
            pltpu.VMEM((chunk * NFEAT,), jnp.int32),
            pltpu.VMEM((chunk * NFEAT, DIM), jnp.float32),
            pltpu.VMEM((chunk * DIM,), jnp.int32),
            pltpu.VMEM((chunk * DIM,), jnp.float32),
            pltpu.VMEM((chunk, DIM), jnp.float32),
            pltpu.VMEM((chunk, L), jnp.float32),
            pltpu.SemaphoreType.DMA,
            pltpu.SemaphoreType.DMA,
        ],
    )(functools.partial(_sc_body, b_per_w=b_per_w, chunk=chunk,
                        npad_u=npad_u, npad_df=npad_df))
    return kern(i, u, df_flat, feat2d, user_flat)


def _finish_body(sums_ref, sq_ref, urow_ref, o_ref):
    uraw = urow_ref[...]
    n2 = jnp.sum(uraw * uraw, axis=1, keepdims=True)
    s = jnp.minimum(1.0, 1.0 / (jnp.sqrt(n2) + EPS))
    uh = uraw * s
    v = sums_ref[...] + uh
    t = (jnp.sum(v * v, axis=1)
         - jnp.sum(sq_ref[...], axis=1)
         - jnp.sum(uh * uh, axis=1))
    o_ref[...] = jax.nn.sigmoid(t)


def _finish(sums, sq, urows):
    B = sums.shape[0]
    blk = 2048
    return pl.pallas_call(
        _finish_body,
        grid=(B // blk,),
        in_specs=[
            pl.BlockSpec((blk, DIM), lambda b: (b, 0)),
            pl.BlockSpec((blk, L), lambda b: (b, 0)),
            pl.BlockSpec((blk, DIM), lambda b: (b, 0)),
        ],
        out_specs=pl.BlockSpec((blk,), lambda b: (b,)),
        out_shape=jax.ShapeDtypeStruct((B,), jnp.float32),
    )(sums, sq, urows)


def kernel(u, i, user_table, item_feat_table, item_df_idx):
    B, d = u.shape[0], user_table.shape[1]
    user_flat, npad_u = _detile_flat(user_table.T)
    df_flat, npad_df = _detile_flat(item_df_idx.T)
    feat2d = _norm_flatten(item_feat_table.T)
    sums, sq, urows_flat = _sc_gather(i, u, df_flat, feat2d, user_flat,
                                      npad_u, npad_df)
    return _finish(sums, sq, urows_flat.reshape(B, d))


# speedup vs baseline: 7.1343x; 1.1028x over previous
import functools

import jax
import jax.numpy as jnp
from jax import lax
from jax.experimental import pallas as pl
from jax.experimental.pallas import tpu as pltpu
from jax.experimental.pallas import tpu_sc as plsc

NFEAT = 26
DIM = 32
EPS = 1e-7

NC = 2
NS = 16
NW = NC * NS
L = 16

FLBLK = 16384


def _cdiv(a, b):
    return -(-a // b)


def _detile_body(x_ref, o_ref, *, din):
    o_ref[...] = x_ref[...].reshape(din, FLBLK // 128, 128)


def _detile_flat(table_t):
    D, N = table_t.shape
    g = _cdiv(N, FLBLK)
    out = pl.pallas_call(
        functools.partial(_detile_body, din=D),
        grid=(g,),
        in_specs=[pl.BlockSpec((D, FLBLK), lambda b: (0, b))],
        out_specs=pl.BlockSpec((D, FLBLK // 128, 128), lambda b: (0, b, 0)),
        out_shape=jax.ShapeDtypeStruct((D, g * FLBLK // 128, 128),
                                       table_t.dtype),
    )(table_t)
    return out.reshape(D * g * FLBLK), g * FLBLK


def _norm_flat_body(x_ref, o_ref):
    x = x_ref[...]
    n2 = jnp.sum(x * x, axis=0, keepdims=True)
    s = jnp.minimum(1.0, 1.0 / (jnp.sqrt(n2) + EPS))
    x3 = (x * s).T.reshape(FLBLK // 4, 4, DIM)
    o_ref[...] = jnp.concatenate([x3[:, m, :] for m in range(4)], axis=1)


def _norm_flatten(table_t):
    D, N = table_t.shape
    g = _cdiv(N, FLBLK)
    out = pl.pallas_call(
        _norm_flat_body,
        grid=(g,),
        in_specs=[pl.BlockSpec((D, FLBLK), lambda b: (0, b))],
        out_specs=pl.BlockSpec((FLBLK // 4, 128), lambda b: (b, 0)),
        out_shape=jax.ShapeDtypeStruct((g * FLBLK // 4, 128), jnp.float32),
    )(table_t)
    return out.reshape(g * FLBLK, DIM)


_DIV26_M = 20165
_DIV26_S = 19


def _sc_body(i_hbm, u_hbm, dfflat_hbm, feat_hbm, userflat_hbm,
             sum_hbm, sq_hbm, urow_hbm,
             i_v, u_v, dfaddr_v, dfidx0_v, dfidx1_v, rows0_v, rows1_v,
             uaddr_v, urows_v, sums_v, sq_v,
             sem_df, sem_u, sem_f, *, b_per_w, chunk, npad_u, npad_df):
    wid = lax.axis_index("s") * NC + lax.axis_index("c")
    nchunks = b_per_w // chunk
    npos = chunk * NFEAT
    ngroups = npos // L
    npieces = npos // 128
    nupieces = chunk * DIM // 128
    iota = lax.iota(jnp.int32, L)
    ulo = iota * npad_u
    uhi = (iota + L) * npad_u
    dfidx = [dfidx0_v, dfidx1_v]
    rows = [rows0_v, rows1_v]

    def stage_and_fire(k, dfidx_b):
        base = wid * b_per_w + k * chunk
        pltpu.sync_copy(i_hbm.at[pl.ds(base, chunk)], i_v)
        pltpu.sync_copy(u_hbm.at[pl.ds(base, chunk)], u_v)

        def g_body(g, carry):
            p = g * L + iota
            c = lax.shift_right_logical(p * _DIV26_M, _DIV26_S)
            f = p - c * NFEAT
            ivals = plsc.load_gather(i_v, [c])
            dfaddr_v[pl.ds(g * L, L)] = f * npad_df + ivals
            return carry

        lax.fori_loop(0, ngroups, g_body, 0)
        for j in range(npieces):
            pltpu.make_async_copy(
                dfflat_hbm.at[dfaddr_v.at[pl.ds(j * 128, 128)]],
                dfidx_b.at[pl.ds(j * 128, 128)], sem_df).start()

        def uaddr_body(c, carry):
            uvals = plsc.load_gather(u_v, [jnp.zeros((L,), jnp.int32) + c])
            uaddr_v[pl.ds(c * DIM, L)] = ulo + uvals
            uaddr_v[pl.ds(c * DIM + L, L)] = uhi + uvals
            return carry

        lax.fori_loop(0, chunk, uaddr_body, 0)
        for j in range(nupieces):
            pltpu.make_async_copy(
                userflat_hbm.at[uaddr_v.at[pl.ds(j * 128, 128)]],
                urows_v.at[pl.ds(j * 128, 128)], sem_u).start()

    def compute_store(k, rows_b):
        base = wid * b_per_w + k * chunk

        def c_body(c, carry):
            r = c * NFEAT
            acc0 = jnp.zeros((L,), jnp.float32)
            acc1 = jnp.zeros((L,), jnp.float32)
            sq = jnp.zeros((L,), jnp.float32)
            for f in range(NFEAT):
                x0 = rows_b[r + f, 0:L]
                x1 = rows_b[r + f, L:DIM]
                acc0 = acc0 + x0
                acc1 = acc1 + x1
                sq = sq + x0 * x0
                sq = sq + x1 * x1
            sums_v[c, 0:L] = acc0
            sums_v[c, L:DIM] = acc1
            sq_v[c, 0:L] = sq
            return carry

        lax.fori_loop(0, chunk, c_body, 0)
        pltpu.sync_copy(sums_v, sum_hbm.at[pl.ds(base, chunk)])
        pltpu.sync_copy(sq_v, sq_hbm.at[pl.ds(base, chunk)])

    stage_and_fire(0, dfidx[0])
    for k in range(nchunks):
        buf = k % 2
        base = wid * b_per_w + k * chunk
        for j in range(npieces):
            pltpu.make_async_copy(
                dfflat_hbm.at[dfaddr_v.at[pl.ds(j * 128, 128)]],
                dfidx[buf].at[pl.ds(j * 128, 128)], sem_df).wait()
        for j in range(npieces):
            pltpu.make_async_copy(
                feat_hbm.at[dfidx[buf].at[pl.ds(j * 128, 128)]],
                rows[buf].at[pl.ds(j * 128, 128)], sem_f).start()
        for j in range(nupieces):
            pltpu.make_async_copy(
                userflat_hbm.at[uaddr_v.at[pl.ds(j * 128, 128)]],
                urows_v.at[pl.ds(j * 128, 128)], sem_u).wait()
        pltpu.sync_copy(urows_v, urow_hbm.at[pl.ds(base * DIM, chunk * DIM)])
        if k + 1 < nchunks:
            stage_and_fire(k + 1, dfidx[1 - buf])
        if k > 0:
            for j in range(npieces):
                pltpu.make_async_copy(
                    feat_hbm.at[dfidx[1 - buf].at[pl.ds(j * 128, 128)]],
                    rows[1 - buf].at[pl.ds(j * 128, 128)], sem_f).wait()
            compute_store(k - 1, rows[1 - buf])
    lastbuf = (nchunks - 1) % 2
    for j in range(npieces):
        pltpu.make_async_copy(
            feat_hbm.at[dfidx[lastbuf].at[pl.ds(j * 128, 128)]],
            rows[lastbuf].at[pl.ds(j * 128, 128)], sem_f).wait()
    compute_store(nchunks - 1, rows[lastbuf])


def _sc_gather(i, u, df_flat, feat2d, user_flat, npad_u, npad_df):
    B = i.shape[0]
    b_per_w = B // NW
    chunk = 64
    mesh = plsc.VectorSubcoreMesh(core_axis_name="c", subcore_axis_name="s")
    kern = functools.partial(
        pl.kernel,
        mesh=mesh,
        compiler_params=pltpu.CompilerParams(
            use_tc_tiling_on_sc=False, needs_layout_passes=False),
        out_type=[
            jax.ShapeDtypeStruct((B, DIM), jnp.float32),
            jax.ShapeDtypeStruct((B, L), jnp.float32),
            jax.ShapeDtypeStruct((B * DIM,), jnp.float32),
        ],
        scratch_types=[
            pltpu.VMEM((chunk,), jnp.int32),
            pltpu.VMEM((chunk,), jnp.int32),
            pltpu.VMEM((chunk * NFEAT,), jnp.int32),
            pltpu.VMEM((chunk * NFEAT,), jnp.int32),
            pltpu.VMEM((chunk * NFEAT,), jnp.int32),
            pltpu.VMEM((chunk * NFEAT, DIM), jnp.float32),
            pltpu.VMEM((chunk * NFEAT, DIM), jnp.float32),
            pltpu.VMEM((chunk * DIM,), jnp.int32),
            pltpu.VMEM((chunk * DIM,), jnp.float32),
            pltpu.VMEM((chunk, DIM), jnp.float32),
            pltpu.VMEM((chunk, L), jnp.float32),
            pltpu.SemaphoreType.DMA,
            pltpu.SemaphoreType.DMA,
            pltpu.SemaphoreType.DMA,
        ],
    )(functools.partial(_sc_body, b_per_w=b_per_w, chunk=chunk,
                        npad_u=npad_u, npad_df=npad_df))
    return kern(i, u, df_flat, feat2d, user_flat)


def _finish_body(sums_ref, sq_ref, urow_ref, o_ref):
    uraw = urow_ref[...]
    n2 = jnp.sum(uraw * uraw, axis=1, keepdims=True)
    s = jnp.minimum(1.0, 1.0 / (jnp.sqrt(n2) + EPS))
    uh = uraw * s
    v = sums_ref[...] + uh
    t = (jnp.sum(v * v, axis=1)
         - jnp.sum(sq_ref[...], axis=1)
         - jnp.sum(uh * uh, axis=1))
    o_ref[...] = jax.nn.sigmoid(t)


def _finish(sums, sq, urows):
    B = sums.shape[0]
    blk = 2048
    return pl.pallas_call(
        _finish_body,
        grid=(B // blk,),
        in_specs=[
            pl.BlockSpec((blk, DIM), lambda b: (b, 0)),
            pl.BlockSpec((blk, L), lambda b: (b, 0)),
            pl.BlockSpec((blk, DIM), lambda b: (b, 0)),
        ],
        out_specs=pl.BlockSpec((blk,), lambda b: (b,)),
        out_shape=jax.ShapeDtypeStruct((B,), jnp.float32),
    )(sums, sq, urows)


def kernel(u, i, user_table, item_feat_table, item_df_idx):
    B, d = u.shape[0], user_table.shape[1]
    user_flat, npad_u = _detile_flat(user_table.T)
    df_flat, npad_df = _detile_flat(item_df_idx.T)
    feat2d = _norm_flatten(item_feat_table.T)
    sums, sq, urows_flat = _sc_gather(i, u, df_flat, feat2d, user_flat,
                                      npad_u, npad_df)
    return _finish(sums, sq, urows_flat.reshape(B, d))
